# baseline (speedup 1.0000x reference)
import functools

import jax
import jax.numpy as jnp
from jax import lax
from jax.experimental import pallas as pl
from jax.experimental.pallas import tpu as pltpu
from jax.experimental.pallas import tpu_sc as plsc

_N = 256
_DT = 0.05
_LANES = 16
_BLK = 128


def _make_kernel(num_blocks: int, num_workers: int):
    wblocks = num_blocks // num_workers
    assert wblocks * num_workers == num_blocks
    mesh = plsc.VectorSubcoreMesh(core_axis_name="c", subcore_axis_name="s")
    nc = mesh.num_cores
    groups = _BLK // _LANES

    nsub = 4
    sub = wblocks // nsub
    assert sub * nsub == wblocks

    @functools.partial(
        pl.kernel,
        out_type=jax.ShapeDtypeStruct((num_blocks, 2, _BLK), jnp.float32),
        mesh=mesh,
        scratch_types=[
            pltpu.VMEM((sub, 2, _BLK), jnp.float32),
            pltpu.VMEM((sub, 2, _BLK), jnp.float32),
            pltpu.VMEM((_N,), jnp.float32),
            pltpu.SemaphoreType.DMA,
            pltpu.SemaphoreType.DMA,
            pltpu.SemaphoreType.DMA,
            pltpu.SemaphoreType.DMA,
        ],
        compiler_params=pltpu.CompilerParams(
            needs_layout_passes=False, use_tc_tiling_on_sc=False
        ),
    )
    def run(x_hbm, f_hbm, out_hbm, bufa, bufb, ftab, ina, inb, outa, outb):
        wid = lax.axis_index("s") * nc + lax.axis_index("c")
        b0 = wid * wblocks
        bufs = (bufa, bufb)
        insem = (ina, inb)
        outsem = (outa, outb)

        def start_in(s):
            return pltpu.async_copy(
                x_hbm.at[pl.ds(b0 + s * sub, sub)], bufs[s % 2], insem[s % 2]
            )

        loads = [start_in(0), start_in(1)]
        pltpu.sync_copy(f_hbm, ftab)
        stores = []
        for s in range(nsub):
            buf = bufs[s % 2]
            loads[s].wait()

            @plsc.parallel_loop(0, sub, 1, unroll=2)
            def step(t):
                for g in range(groups):
                    xg = buf[t, 0, pl.ds(g * _LANES, _LANES)]
                    v = xg * 128.0 + 128.5
                    v = jnp.minimum(jnp.maximum(v, 0.0), 255.0)
                    idx = v.astype(jnp.int32)
                    fv = plsc.load_gather(ftab, [idx])
                    vs = buf.at[t, 1, pl.ds(g * _LANES, _LANES)]
                    plsc.addupdate(vs, fv * _DT)

            stores.append(
                pltpu.async_copy(
                    buf, out_hbm.at[pl.ds(b0 + s * sub, sub)], outsem[s % 2]
                )
            )
            if s + 2 < nsub:
                stores[s].wait()
                loads.append(start_in(s + 2))
        stores[nsub - 2].wait()
        stores[nsub - 1].wait()

    return run


def kernel(X, force):
    b = X.shape[0]
    xt = jnp.transpose(jnp.reshape(X, (b // _BLK, _BLK, 2)), (0, 2, 1))
    yt = _make_kernel(b // _BLK, 32)(xt, force.astype(jnp.float32))
    return jnp.reshape(jnp.transpose(yt, (0, 2, 1)), (b, 2))


# speedup vs baseline: 1.0750x; 1.0750x over previous
import functools

import jax
import jax.numpy as jnp
from jax import lax
from jax.experimental import pallas as pl
from jax.experimental.pallas import tpu as pltpu
from jax.experimental.pallas import tpu_sc as plsc

_N = 256
_DT = 0.05
_LANES = 16
_BLK = 128


def _make_kernel(num_blocks: int, num_workers: int):
    wblocks = num_blocks // num_workers
    assert wblocks * num_workers == num_blocks
    mesh = plsc.VectorSubcoreMesh(core_axis_name="c", subcore_axis_name="s")
    nc = mesh.num_cores
    groups = _BLK // _LANES

    nsub = 2
    sub = wblocks // nsub
    assert sub * nsub == wblocks

    @functools.partial(
        pl.kernel,
        out_type=jax.ShapeDtypeStruct((num_blocks, 2, _BLK), jnp.float32),
        mesh=mesh,
        scratch_types=[
            pltpu.VMEM((sub, 2, _BLK), jnp.float32),
            pltpu.VMEM((sub, 2, _BLK), jnp.float32),
            pltpu.VMEM((_N,), jnp.float32),
            pltpu.SemaphoreType.DMA,
            pltpu.SemaphoreType.DMA,
            pltpu.SemaphoreType.DMA,
            pltpu.SemaphoreType.DMA,
        ],
        compiler_params=pltpu.CompilerParams(
            needs_layout_passes=False, use_tc_tiling_on_sc=False
        ),
    )
    def run(x_hbm, f_hbm, out_hbm, bufa, bufb, ftab, ina, inb, outa, outb):
        wid = lax.axis_index("s") * nc + lax.axis_index("c")
        b0 = wid * wblocks
        bufs = (bufa, bufb)
        insem = (ina, inb)
        outsem = (outa, outb)

        def start_in(s):
            return pltpu.async_copy(
                x_hbm.at[pl.ds(b0 + s * sub, sub)], bufs[s % 2], insem[s % 2]
            )

        loads = [start_in(0), start_in(1)]
        pltpu.sync_copy(f_hbm, ftab)
        stores = []
        for s in range(nsub):
            buf = bufs[s % 2]
            loads[s].wait()

            @plsc.parallel_loop(0, sub, 1, unroll=2)
            def step(t):
                for g in range(groups):
                    xg = buf[t, 0, pl.ds(g * _LANES, _LANES)]
                    v = xg * 128.0 + 128.5
                    v = jnp.minimum(jnp.maximum(v, 0.0), 255.0)
                    idx = v.astype(jnp.int32)
                    fv = plsc.load_gather(ftab, [idx])
                    vs = buf.at[t, 1, pl.ds(g * _LANES, _LANES)]
                    plsc.addupdate(vs, fv * _DT)

            stores.append(
                pltpu.async_copy(
                    buf, out_hbm.at[pl.ds(b0 + s * sub, sub)], outsem[s % 2]
                )
            )
            if s + 2 < nsub:
                stores[s].wait()
                loads.append(start_in(s + 2))
        stores[nsub - 2].wait()
        stores[nsub - 1].wait()

    return run


def kernel(X, force):
    b = X.shape[0]
    xt = jnp.transpose(jnp.reshape(X, (b // _BLK, _BLK, 2)), (0, 2, 1))
    yt = _make_kernel(b // _BLK, 32)(xt, force.astype(jnp.float32))
    return jnp.reshape(jnp.transpose(yt, (0, 2, 1)), (b, 2))
